# Initial kernel scaffold; baseline (speedup 1.0000x reference)
import jax
import jax.numpy as jnp
from jax.experimental import pallas as pl


def kernel(cell_feat, zeta, track_feat, c2n, t2n, n2n, cell_W0, cell_b0, cell_g, cell_beta, cell_W1, cell_b1, cell_W2, cell_b2, track_W0, track_b0, track_g, track_beta, track_W1, track_b1, track_W2, track_b2, node0_W0, node0_b0, node0_W1, node0_b1, node0_W2, node0_b2, node1_W0, node1_b0, node1_W1, node1_b1, node1_W2, node1_b2):
    raise NotImplementedError("write your pallas kernel here")


# speedup vs baseline: 2.8195x; 2.8195x over previous
import functools

import jax
import jax.numpy as jnp
from jax import lax
from jax.experimental import pallas as pl
from jax.experimental.pallas import tpu as pltpu
from jax.experimental.pallas import tpu_sc as plsc

N_NODES = 10000
H = 128
NC = 2
NS = 16
NW = NC * NS
CHUNK = 128
ACC_ROWS = 10112
RPS = ACC_ROWS // NS


@functools.partial(jax.jit, static_argnames=("n_chunks",))
def _sc_segsum(src2d, dst2d, table, zslab, *, n_chunks):
    mesh = plsc.VectorSubcoreMesh(core_axis_name="c", subcore_axis_name="s")

    @functools.partial(
        pl.kernel,
        out_type=jax.ShapeDtypeStruct((NC, ACC_ROWS, H), jnp.float32),
        mesh=mesh,
        scratch_types=[
            pltpu.VMEM((n_chunks, CHUNK), jnp.int32),
            pltpu.VMEM((n_chunks, CHUNK), jnp.int32),
            pltpu.VMEM((CHUNK, H), jnp.float32),
            pltpu.VMEM_SHARED((ACC_ROWS, H), jnp.float32),
            pltpu.SemaphoreType.DMA,
        ],
    )
    def seg_kernel(src_hbm, dst_hbm, table_hbm, zs_hbm, out_hbm,
                   srcv, dstv, rows, acc, sem):
        c = lax.axis_index("c")
        s = lax.axis_index("s")
        w = s * NC + c
        pltpu.sync_copy(zs_hbm, acc.at[pl.ds(s * RPS, RPS)])
        pltpu.sync_copy(src_hbm.at[w], srcv)
        pltpu.sync_copy(dst_hbm.at[w], dstv)
        plsc.subcore_barrier()

        def body(i, carry):
            pltpu.async_copy(table_hbm.at[srcv.at[i]], rows, sem).wait()
            pltpu.sync_copy(rows, acc.at[dstv.at[i]], add=True)
            return carry

        lax.fori_loop(0, n_chunks, body, 0)
        plsc.subcore_barrier()
        pltpu.sync_copy(acc.at[pl.ds(s * RPS, RPS)],
                        out_hbm.at[c, pl.ds(s * RPS, RPS)])

    return seg_kernel(src2d, dst2d, table, zslab)


def _hid_stats_body(x_ref, w0_ref, b0_ref, h_ref, st_ref):
    i = pl.program_id(0)
    h = jnp.maximum(x_ref[...] @ w0_ref[...] + b0_ref[...], 0.0)
    h_ref[...] = h
    s = jnp.sum(h, axis=0, keepdims=True)
    ss = jnp.sum(h * h, axis=0, keepdims=True)
    st = jnp.concatenate([s, ss], axis=0)

    @pl.when(i == 0)
    def _():
        st_ref[...] = jnp.zeros_like(st_ref)

    st_ref[...] += st


def _bn_finish_body(n_rows, h_ref, st_ref, g_ref, beta_ref, w1_ref, b1_ref,
                    w2_ref, b2_ref, out_ref):
    mu = st_ref[0:1, :] / n_rows
    var = st_ref[1:2, :] / n_rows - mu * mu
    hn = (h_ref[...] - mu) * jax.lax.rsqrt(var + 1e-5) * g_ref[...] + beta_ref[...]
    h2 = jnp.maximum(hn @ w1_ref[...] + b1_ref[...], 0.0)
    out_ref[...] = h2 @ w2_ref[...] + b2_ref[...]


def _mlp_bn(x, W0, b0, g, beta, W1, b1, W2, b2, blk):
    n, d = x.shape
    grid = n // blk
    h, st = pl.pallas_call(
        _hid_stats_body,
        grid=(grid,),
        in_specs=[
            pl.BlockSpec((blk, d), lambda i: (i, 0)),
            pl.BlockSpec((d, H), lambda i: (0, 0)),
            pl.BlockSpec((1, H), lambda i: (0, 0)),
        ],
        out_specs=[
            pl.BlockSpec((blk, H), lambda i: (i, 0)),
            pl.BlockSpec((2, H), lambda i: (0, 0)),
        ],
        out_shape=[
            jax.ShapeDtypeStruct((n, H), jnp.float32),
            jax.ShapeDtypeStruct((2, H), jnp.float32),
        ],
    )(x, W0, b0.reshape(1, H))
    out = pl.pallas_call(
        functools.partial(_bn_finish_body, float(n)),
        grid=(grid,),
        in_specs=[
            pl.BlockSpec((blk, H), lambda i: (i, 0)),
            pl.BlockSpec((2, H), lambda i: (0, 0)),
            pl.BlockSpec((1, H), lambda i: (0, 0)),
            pl.BlockSpec((1, H), lambda i: (0, 0)),
            pl.BlockSpec((H, H), lambda i: (0, 0)),
            pl.BlockSpec((1, H), lambda i: (0, 0)),
            pl.BlockSpec((H, H), lambda i: (0, 0)),
            pl.BlockSpec((1, H), lambda i: (0, 0)),
        ],
        out_specs=pl.BlockSpec((blk, H), lambda i: (i, 0)),
        out_shape=jax.ShapeDtypeStruct((n, H), jnp.float32),
    )(h, st, g.reshape(1, H), beta.reshape(1, H), W1, b1.reshape(1, H),
      W2, b2.reshape(1, H))
    return out


def _add_colsum_body(pa_ref, pb_ref, out_ref, cs_ref):
    i = pl.program_id(0)
    sm = pa_ref[0] + pb_ref[0]
    out_ref[...] = sm

    @pl.when(i == 0)
    def _():
        cs_ref[...] = jnp.zeros_like(cs_ref)

    cs_ref[...] += jnp.sum(sm, axis=0, keepdims=True)


def _add_colsum(partials, blk):
    grid = N_NODES // blk
    return pl.pallas_call(
        _add_colsum_body,
        grid=(grid,),
        in_specs=[
            pl.BlockSpec((1, blk, H), lambda i: (0, i, 0)),
            pl.BlockSpec((1, blk, H), lambda i: (1, i, 0)),
        ],
        out_specs=[
            pl.BlockSpec((blk, H), lambda i: (i, 0)),
            pl.BlockSpec((1, H), lambda i: (0, 0)),
        ],
        out_shape=[
            jax.ShapeDtypeStruct((N_NODES, H), jnp.float32),
            jax.ShapeDtypeStruct((1, H), jnp.float32),
        ],
    )(partials, partials)


def _iter_body(pa_ref, pb_ref, nh_ref, cs_ref, w0m_ref, w0h_ref, w0g_ref,
               b0_ref, w1_ref, b1_ref, w2_ref, b2_ref, out_ref, cso_ref):
    i = pl.program_id(0)
    gv = cs_ref[...]
    gv = gv / jnp.maximum(jnp.sqrt(jnp.sum(gv * gv)), 1e-8)
    gterm = gv @ w0g_ref[...] + b0_ref[...]
    msg = pa_ref[0] + pb_ref[0]
    x1 = jnp.maximum(msg @ w0m_ref[...] + nh_ref[...] @ w0h_ref[...] + gterm, 0.0)
    x2 = jnp.maximum(x1 @ w1_ref[...] + b1_ref[...], 0.0)
    o = x2 @ w2_ref[...] + b2_ref[...]
    nrm = jnp.sqrt(jnp.sum(o * o, axis=1, keepdims=True))
    o = o / jnp.maximum(nrm, 1e-8)
    out_ref[...] = o

    @pl.when(i == 0)
    def _():
        cso_ref[...] = jnp.zeros_like(cso_ref)

    cso_ref[...] += jnp.sum(o, axis=0, keepdims=True)


def _iter_update(partials, node_h, cs, W0, b0, W1, b1, W2, b2, blk):
    grid = N_NODES // blk
    W0m, W0h, W0g = W0[0:H], W0[H:2 * H], W0[2 * H:3 * H]
    return pl.pallas_call(
        _iter_body,
        grid=(grid,),
        in_specs=[
            pl.BlockSpec((1, blk, H), lambda i: (0, i, 0)),
            pl.BlockSpec((1, blk, H), lambda i: (1, i, 0)),
            pl.BlockSpec((blk, H), lambda i: (i, 0)),
            pl.BlockSpec((1, H), lambda i: (0, 0)),
            pl.BlockSpec((H, H), lambda i: (0, 0)),
            pl.BlockSpec((H, H), lambda i: (0, 0)),
            pl.BlockSpec((H, H), lambda i: (0, 0)),
            pl.BlockSpec((1, H), lambda i: (0, 0)),
            pl.BlockSpec((H, H), lambda i: (0, 0)),
            pl.BlockSpec((1, H), lambda i: (0, 0)),
            pl.BlockSpec((H, H), lambda i: (0, 0)),
            pl.BlockSpec((1, H), lambda i: (0, 0)),
        ],
        out_specs=[
            pl.BlockSpec((blk, H), lambda i: (i, 0)),
            pl.BlockSpec((1, H), lambda i: (0, 0)),
        ],
        out_shape=[
            jax.ShapeDtypeStruct((N_NODES, H), jnp.float32),
            jax.ShapeDtypeStruct((1, H), jnp.float32),
        ],
    )(partials, partials, node_h, cs, W0m, W0h, W0g, b0.reshape(1, H), W1,
      b1.reshape(1, H), W2, b2.reshape(1, H))


def _pad_edges(src, dst, n_pad_total):
    e = src.shape[0]
    pad = n_pad_total - e
    src = jnp.concatenate([src, jnp.zeros((pad,), jnp.int32)])
    dst = jnp.concatenate([dst, jnp.full((pad,), N_NODES, jnp.int32)])
    nc = n_pad_total // (NW * CHUNK)
    return (src.reshape(NW, nc, CHUNK), dst.reshape(NW, nc, CHUNK), nc)


def kernel(cell_feat, zeta, track_feat, c2n, t2n, n2n, cell_W0, cell_b0,
           cell_g, cell_beta, cell_W1, cell_b1, cell_W2, cell_b2, track_W0,
           track_b0, track_g, track_beta, track_W1, track_b1, track_W2,
           track_b2, node0_W0, node0_b0, node0_W1, node0_b1, node0_W2,
           node0_b2, node1_W0, node1_b0, node1_W1, node1_b1, node1_W2,
           node1_b2):
    zslab = jnp.zeros((RPS, H), jnp.float32)

    x_cell = jnp.concatenate([cell_feat, zeta[:, None]], axis=1)
    cell_h = _mlp_bn(x_cell, cell_W0, cell_b0, cell_g, cell_beta, cell_W1,
                     cell_b1, cell_W2, cell_b2, blk=1000)
    track_h = _mlp_bn(track_feat, track_W0, track_b0, track_g, track_beta,
                      track_W1, track_b1, track_W2, track_b2, blk=1000)

    table0 = jnp.concatenate([cell_h, track_h], axis=0)
    src0 = jnp.concatenate([c2n[0], t2n[0] + cell_h.shape[0]])
    dst0 = jnp.concatenate([c2n[1], t2n[1]])
    src0_2d, dst0_2d, nc0 = _pad_edges(src0, dst0, 12288)
    partials0 = _sc_segsum(src0_2d, dst0_2d, table0, zslab, n_chunks=nc0)
    node_h, cs = _add_colsum(partials0, blk=1000)

    srcn_2d, dstn_2d, ncn = _pad_edges(n2n[0], n2n[1], 327680)
    blocks = [(node0_W0, node0_b0, node0_W1, node0_b1, node0_W2, node0_b2),
              (node1_W0, node1_b0, node1_W1, node1_b1, node1_W2, node1_b2)]
    for bi in range(2):
        W0, b0, W1, b1, W2, b2 = blocks[bi]
        for _ in range(2):
            partials = _sc_segsum(srcn_2d, dstn_2d, node_h, zslab,
                                  n_chunks=ncn)
            node_h, cs = _iter_update(partials, node_h, cs, W0, b0, W1, b1,
                                      W2, b2, blk=1000)
    return node_h


# speedup vs baseline: 3.0267x; 1.0735x over previous
import functools

import jax
import jax.numpy as jnp
from jax import lax
from jax.experimental import pallas as pl
from jax.experimental.pallas import tpu as pltpu
from jax.experimental.pallas import tpu_sc as plsc

N_NODES = 10000
H = 128
NC = 2
NS = 16
NW = NC * NS
CHUNK = 128
ACC_ROWS = 10112
RPS = ACC_ROWS // NS


@functools.partial(jax.jit, static_argnames=("n_chunks",))
def _sc_segsum(packed, table, zslab, *, n_chunks):
    mesh = plsc.VectorSubcoreMesh(core_axis_name="c", subcore_axis_name="s")

    nbuf = 2
    assert n_chunks % nbuf == 0

    @functools.partial(
        pl.kernel,
        out_type=jax.ShapeDtypeStruct((NC, ACC_ROWS, H), jnp.float32),
        mesh=mesh,
        scratch_types=[
            pltpu.VMEM((n_chunks, CHUNK), jnp.int32),
            pltpu.VMEM((nbuf, CHUNK), jnp.int32),
            pltpu.VMEM((nbuf, CHUNK), jnp.int32),
            pltpu.VMEM((nbuf, CHUNK, H), jnp.float32),
            pltpu.VMEM_SHARED((ACC_ROWS, H), jnp.float32),
            [pltpu.SemaphoreType.DMA] * nbuf,
        ],
    )
    def seg_kernel(pk_hbm, table_hbm, zs_hbm, out_hbm,
                   pkv, srcu, dstu, rows, acc, sems):
        c = lax.axis_index("c")
        s = lax.axis_index("s")
        w = s * NC + c
        pltpu.sync_copy(zs_hbm, acc.at[pl.ds(s * RPS, RPS)])
        pltpu.sync_copy(pk_hbm.at[w], pkv)
        plsc.subcore_barrier()

        def unpack(i, b):
            for k in range(CHUNK // 16):
                v = pkv[i, pl.ds(k * 16, 16)]
                srcu[b, pl.ds(k * 16, 16)] = lax.bitwise_and(v, 0xFFFF)
                dstu[b, pl.ds(k * 16, 16)] = lax.shift_right_logical(v, 16)

        for b in range(nbuf):
            unpack(b, b)
            pltpu.async_copy(table_hbm.at[srcu.at[b]], rows.at[b], sems[b])

        def body(j, carry):
            for b in range(nbuf):
                i = nbuf * j + b
                pltpu.make_async_copy(table_hbm.at[srcu.at[b]], rows.at[b],
                                      sems[b]).wait()
                pltpu.sync_copy(rows.at[b], acc.at[dstu.at[b]], add=True)
                unpack(i + nbuf, b)
                pltpu.async_copy(table_hbm.at[srcu.at[b]], rows.at[b],
                                 sems[b])
            return carry

        lax.fori_loop(0, n_chunks // nbuf - 1, body, 0)
        for b in range(nbuf):
            pltpu.make_async_copy(table_hbm.at[srcu.at[b]], rows.at[b],
                                  sems[b]).wait()
            pltpu.sync_copy(rows.at[b], acc.at[dstu.at[b]], add=True)
        plsc.subcore_barrier()
        pltpu.sync_copy(acc.at[pl.ds(s * RPS, RPS)],
                        out_hbm.at[c, pl.ds(s * RPS, RPS)])

    return seg_kernel(packed, table, zslab)


def _hid_stats_body(x_ref, w0_ref, b0_ref, h_ref, st_ref):
    i = pl.program_id(0)
    h = jnp.maximum(x_ref[...] @ w0_ref[...] + b0_ref[...], 0.0)
    h_ref[...] = h
    s = jnp.sum(h, axis=0, keepdims=True)
    ss = jnp.sum(h * h, axis=0, keepdims=True)
    st = jnp.concatenate([s, ss], axis=0)

    @pl.when(i == 0)
    def _():
        st_ref[...] = jnp.zeros_like(st_ref)

    st_ref[...] += st


def _bn_finish_body(n_rows, h_ref, st_ref, g_ref, beta_ref, w1_ref, b1_ref,
                    w2_ref, b2_ref, out_ref):
    mu = st_ref[0:1, :] / n_rows
    var = st_ref[1:2, :] / n_rows - mu * mu
    hn = (h_ref[...] - mu) * jax.lax.rsqrt(var + 1e-5) * g_ref[...] + beta_ref[...]
    h2 = jnp.maximum(hn @ w1_ref[...] + b1_ref[...], 0.0)
    out_ref[...] = h2 @ w2_ref[...] + b2_ref[...]


def _mlp_bn(x, W0, b0, g, beta, W1, b1, W2, b2, blk):
    n, d = x.shape
    grid = n // blk
    h, st = pl.pallas_call(
        _hid_stats_body,
        grid=(grid,),
        in_specs=[
            pl.BlockSpec((blk, d), lambda i: (i, 0)),
            pl.BlockSpec((d, H), lambda i: (0, 0)),
            pl.BlockSpec((1, H), lambda i: (0, 0)),
        ],
        out_specs=[
            pl.BlockSpec((blk, H), lambda i: (i, 0)),
            pl.BlockSpec((2, H), lambda i: (0, 0)),
        ],
        out_shape=[
            jax.ShapeDtypeStruct((n, H), jnp.float32),
            jax.ShapeDtypeStruct((2, H), jnp.float32),
        ],
    )(x, W0, b0.reshape(1, H))
    out = pl.pallas_call(
        functools.partial(_bn_finish_body, float(n)),
        grid=(grid,),
        in_specs=[
            pl.BlockSpec((blk, H), lambda i: (i, 0)),
            pl.BlockSpec((2, H), lambda i: (0, 0)),
            pl.BlockSpec((1, H), lambda i: (0, 0)),
            pl.BlockSpec((1, H), lambda i: (0, 0)),
            pl.BlockSpec((H, H), lambda i: (0, 0)),
            pl.BlockSpec((1, H), lambda i: (0, 0)),
            pl.BlockSpec((H, H), lambda i: (0, 0)),
            pl.BlockSpec((1, H), lambda i: (0, 0)),
        ],
        out_specs=pl.BlockSpec((blk, H), lambda i: (i, 0)),
        out_shape=jax.ShapeDtypeStruct((n, H), jnp.float32),
    )(h, st, g.reshape(1, H), beta.reshape(1, H), W1, b1.reshape(1, H),
      W2, b2.reshape(1, H))
    return out


def _add_colsum_body(pa_ref, pb_ref, out_ref, cs_ref):
    i = pl.program_id(0)
    sm = pa_ref[0] + pb_ref[0]
    out_ref[...] = sm

    @pl.when(i == 0)
    def _():
        cs_ref[...] = jnp.zeros_like(cs_ref)

    cs_ref[...] += jnp.sum(sm, axis=0, keepdims=True)


def _add_colsum(partials, blk):
    grid = N_NODES // blk
    return pl.pallas_call(
        _add_colsum_body,
        grid=(grid,),
        in_specs=[
            pl.BlockSpec((1, blk, H), lambda i: (0, i, 0)),
            pl.BlockSpec((1, blk, H), lambda i: (1, i, 0)),
        ],
        out_specs=[
            pl.BlockSpec((blk, H), lambda i: (i, 0)),
            pl.BlockSpec((1, H), lambda i: (0, 0)),
        ],
        out_shape=[
            jax.ShapeDtypeStruct((N_NODES, H), jnp.float32),
            jax.ShapeDtypeStruct((1, H), jnp.float32),
        ],
    )(partials, partials)


def _iter_body(pa_ref, pb_ref, nh_ref, cs_ref, w0m_ref, w0h_ref, w0g_ref,
               b0_ref, w1_ref, b1_ref, w2_ref, b2_ref, out_ref, cso_ref):
    i = pl.program_id(0)
    gv = cs_ref[...]
    gv = gv / jnp.maximum(jnp.sqrt(jnp.sum(gv * gv)), 1e-8)
    gterm = gv @ w0g_ref[...] + b0_ref[...]
    msg = pa_ref[0] + pb_ref[0]
    x1 = jnp.maximum(msg @ w0m_ref[...] + nh_ref[...] @ w0h_ref[...] + gterm, 0.0)
    x2 = jnp.maximum(x1 @ w1_ref[...] + b1_ref[...], 0.0)
    o = x2 @ w2_ref[...] + b2_ref[...]
    nrm = jnp.sqrt(jnp.sum(o * o, axis=1, keepdims=True))
    o = o / jnp.maximum(nrm, 1e-8)
    out_ref[...] = o

    @pl.when(i == 0)
    def _():
        cso_ref[...] = jnp.zeros_like(cso_ref)

    cso_ref[...] += jnp.sum(o, axis=0, keepdims=True)


def _iter_update(partials, node_h, cs, W0, b0, W1, b1, W2, b2, blk):
    grid = N_NODES // blk
    W0m, W0h, W0g = W0[0:H], W0[H:2 * H], W0[2 * H:3 * H]
    return pl.pallas_call(
        _iter_body,
        grid=(grid,),
        in_specs=[
            pl.BlockSpec((1, blk, H), lambda i: (0, i, 0)),
            pl.BlockSpec((1, blk, H), lambda i: (1, i, 0)),
            pl.BlockSpec((blk, H), lambda i: (i, 0)),
            pl.BlockSpec((1, H), lambda i: (0, 0)),
            pl.BlockSpec((H, H), lambda i: (0, 0)),
            pl.BlockSpec((H, H), lambda i: (0, 0)),
            pl.BlockSpec((H, H), lambda i: (0, 0)),
            pl.BlockSpec((1, H), lambda i: (0, 0)),
            pl.BlockSpec((H, H), lambda i: (0, 0)),
            pl.BlockSpec((1, H), lambda i: (0, 0)),
            pl.BlockSpec((H, H), lambda i: (0, 0)),
            pl.BlockSpec((1, H), lambda i: (0, 0)),
        ],
        out_specs=[
            pl.BlockSpec((blk, H), lambda i: (i, 0)),
            pl.BlockSpec((1, H), lambda i: (0, 0)),
        ],
        out_shape=[
            jax.ShapeDtypeStruct((N_NODES, H), jnp.float32),
            jax.ShapeDtypeStruct((1, H), jnp.float32),
        ],
    )(partials, partials, node_h, cs, W0m, W0h, W0g, b0.reshape(1, H), W1,
      b1.reshape(1, H), W2, b2.reshape(1, H))


def _pad_edges(src, dst, n_pad_total):
    e = src.shape[0]
    pad = n_pad_total - e
    src = jnp.concatenate([src, jnp.zeros((pad,), jnp.int32)])
    dst = jnp.concatenate([dst, jnp.full((pad,), N_NODES, jnp.int32)])
    packed = src | (dst << 16)
    nc = n_pad_total // (NW * CHUNK)
    return packed.reshape(NW, nc, CHUNK), nc


def kernel(cell_feat, zeta, track_feat, c2n, t2n, n2n, cell_W0, cell_b0,
           cell_g, cell_beta, cell_W1, cell_b1, cell_W2, cell_b2, track_W0,
           track_b0, track_g, track_beta, track_W1, track_b1, track_W2,
           track_b2, node0_W0, node0_b0, node0_W1, node0_b1, node0_W2,
           node0_b2, node1_W0, node1_b0, node1_W1, node1_b1, node1_W2,
           node1_b2):
    zslab = jnp.zeros((RPS, H), jnp.float32)

    x_cell = jnp.concatenate([cell_feat, zeta[:, None]], axis=1)
    cell_h = _mlp_bn(x_cell, cell_W0, cell_b0, cell_g, cell_beta, cell_W1,
                     cell_b1, cell_W2, cell_b2, blk=1000)
    track_h = _mlp_bn(track_feat, track_W0, track_b0, track_g, track_beta,
                      track_W1, track_b1, track_W2, track_b2, blk=1000)

    table0 = jnp.concatenate([cell_h, track_h], axis=0)
    src0 = jnp.concatenate([c2n[0], t2n[0] + cell_h.shape[0]])
    dst0 = jnp.concatenate([c2n[1], t2n[1]])
    pk0, nc0 = _pad_edges(src0, dst0, 16384)
    partials0 = _sc_segsum(pk0, table0, zslab, n_chunks=nc0)
    node_h, cs = _add_colsum(partials0, blk=1000)

    pkn, ncn = _pad_edges(n2n[0], n2n[1], 327680)
    blocks = [(node0_W0, node0_b0, node0_W1, node0_b1, node0_W2, node0_b2),
              (node1_W0, node1_b0, node1_W1, node1_b1, node1_W2, node1_b2)]
    for bi in range(2):
        W0, b0, W1, b1, W2, b2 = blocks[bi]
        for _ in range(2):
            partials = _sc_segsum(pkn, node_h, zslab, n_chunks=ncn)
            node_h, cs = _iter_update(partials, node_h, cs, W0, b0, W1, b1,
                                      W2, b2, blk=1000)
    return node_h
